# baseline (speedup 1.0000x reference)
import functools

import jax
import jax.numpy as jnp
from jax import lax
from jax.experimental import pallas as pl
from jax.experimental.pallas import tpu as pltpu
from jax.experimental.pallas import tpu_sc as plsc

_GAMMA = 10.0
_NA = 50000
_CHUNK = 1568
_PIECE = 112
_LUTB = 384
_LUTROWS = 392


def _sc_atom(ia_h, ic_h, id_h, iq_h, m_h, ta_h, lut_h, out_h,
             ia_v, ic_v, id_v, iq_v, m_v, ta_v, lut_v, ob0, ob1, sem0, sem1):
    w = lax.axis_index("s") * 2 + lax.axis_index("c")
    base = jnp.minimum(w * _CHUNK, _NA - _CHUNK)
    pltpu.sync_copy(ta_h, ta_v)
    pltpu.sync_copy(lut_h, lut_v)
    pltpu.sync_copy(ia_h.at[pl.ds(base, _CHUNK)], ia_v)
    pltpu.sync_copy(ic_h.at[pl.ds(base, _CHUNK)], ic_v)
    pltpu.sync_copy(id_h.at[pl.ds(base, _CHUNK)], id_v)
    pltpu.sync_copy(iq_h.at[pl.ds(base, _CHUNK)], iq_v)
    pltpu.sync_copy(m_h.at[pl.ds(base, _CHUNK)], m_v)
    lane = lax.iota(jnp.int32, 16)
    copies = []
    for p in range(_CHUNK // _PIECE):
        ob = ob0 if p % 2 == 0 else ob1
        sem = sem0 if p % 2 == 0 else sem1
        if p >= 2:
            copies[p - 2].wait()

        def group(g, carry, p=p, ob=ob):
            off = p * _PIECE + g * 16
            idx16 = lane + off
            ia16 = plsc.load_gather(ia_v, [idx16])
            ic16 = plsc.load_gather(ic_v, [idx16])
            id16 = plsc.load_gather(id_v, [idx16])
            iq16 = plsc.load_gather(iq_v, [idx16])
            m16 = plsc.load_gather(m_v, [idx16])
            abase = ia16 * 128
            cdbase = (119 + ic16 * 12 + id16) * 128
            qbase = (215 + iq16) * 128
            xs = m16 * (_LUTB / 20.0)
            qq = jnp.maximum(jnp.minimum(xs.astype(jnp.int32), _LUTB - 1), 0)
            fr = xs - qq.astype(jnp.float32)
            lbase = qq * 128
            row = g * 16 + lane

            def colstep(j, c):
                acc = plsc.load_gather(ta_v, [abase + j])
                acc = acc + plsc.load_gather(ta_v, [cdbase + j])
                acc = acc + plsc.load_gather(ta_v, [qbase + j])
                la = plsc.load_gather(lut_v, [lbase + j])
                lb = plsc.load_gather(lut_v, [lbase + 128 + j])
                acc = acc + la + fr * (lb - la)
                plsc.store_scatter(ob, [row, lane * 0 + j], acc)
                return c

            lax.fori_loop(0, 128, colstep, 0, unroll=4)
            return carry

        lax.fori_loop(0, _PIECE // 16, group, 0)
        copies.append(pltpu.async_copy(
            ob, out_h.at[pl.ds(base + p * _PIECE, _PIECE)], sem))
    copies[-2].wait()
    copies[-1].wait()


def _prep_body(Wb, tc, td, lut, tcd):
    colr = lax.broadcasted_iota(jnp.int32, (24, 1), 0).astype(jnp.float32)
    ctr = jnp.where(colr < 20.0, colr, -1e6)
    xs = lax.broadcasted_iota(jnp.int32, (1, _LUTROWS), 1).astype(jnp.float32)
    xs = xs * (20.0 / _LUTB)
    F = jnp.where(colr == 20.0, 1.0, jnp.exp(-_GAMMA * (xs - ctr) ** 2))
    lut[...] = lax.dot_general(F, Wb[...], (((0,), (0,)), ((), ())),
                               preferred_element_type=jnp.float32)
    r = lax.broadcasted_iota(jnp.int32, (1, 96), 1)
    rc = r // 12
    rd = r - rc * 12
    ohc = (lax.broadcasted_iota(jnp.int32, (8, 1), 0) == rc).astype(jnp.float32)
    ohd = (lax.broadcasted_iota(jnp.int32, (16, 1), 0) == rd).astype(jnp.float32)
    tcd[...] = (
        lax.dot_general(ohc, tc[...], (((0,), (0,)), ((), ())),
                        preferred_element_type=jnp.float32)
        + lax.dot_general(ohd, td[...], (((0,), (0,)), ((), ())),
                          preferred_element_type=jnp.float32))


def _bond_body(it, ist, ij, ln, Tab, Tba, oab, oba):
    one = jnp.bfloat16(1.0)
    zero = jnp.bfloat16(0.0)
    colh = lax.broadcasted_iota(jnp.int32, (16, 1), 0).astype(jnp.bfloat16)
    itb = it[0, 0, :][None, :].astype(jnp.bfloat16)
    isb = ist[0, 0, :][None, :].astype(jnp.bfloat16) + 8
    FH = jnp.where((colh == itb) | (colh == isb), one, zero)
    colr = lax.broadcasted_iota(jnp.int32, (32, 1), 0).astype(jnp.float32) + 16.0
    ctr = jnp.where((colr >= 20.0) & (colr < 40.0), (colr - 20.0) * 0.1, -1e6)
    x = ln[0, 0, :][None, :]
    t = (-_GAMMA * (x - ctr) ** 2).astype(jnp.bfloat16)
    hot = (colr == ij[0, 0, :][None, :].astype(jnp.float32) + 16.0) | (colr == 40.0)
    FR = jnp.where(hot, one, jnp.exp(t))
    F = jnp.concatenate([FH, FR], axis=0)
    oab[...] = lax.dot_general(F, Tab[...], (((0,), (0,)), ((), ())),
                               preferred_element_type=jnp.float32)
    oba[...] = lax.dot_general(F, Tba[...], (((0,), (0,)), ((), ())),
                               preferred_element_type=jnp.float32)


def _angle_body(ang, T, o):
    one = jnp.bfloat16(1.0)
    col = lax.broadcasted_iota(jnp.int32, (40, 1), 0).astype(jnp.float32)
    ctr = jnp.where(col < 32.0, col * 0.1, -1e6)
    x = ang[0, 0, :][None, :]
    t = (-_GAMMA * (x - ctr) ** 2).astype(jnp.bfloat16)
    F = jnp.where(col == 32.0, one, jnp.exp(t))
    o[...] = lax.dot_general(F, T[...], (((0,), (0,)), ((), ())),
                             preferred_element_type=jnp.float32)


def _vec_spec(B):
    return pl.BlockSpec((1, 1, B), lambda i: (i, 0, 0))


def _tab_spec(shape):
    return pl.BlockSpec(shape, lambda i: (0, 0))


def kernel(idx_atomic, idx_chir, idx_deg, idx_charge, atom_mass, idx_btype,
           idx_bstereo, idx_bconj, bond_length, ab_edge_index, bond_angle,
           ba_edge_index, t_atomic, t_chir, t_deg, t_charge, W_mass, b_mass,
           tb_type_ab, tb_stereo_ab, tb_conj_ab, W_len_ab, b_len_ab,
           tb_type_ba, tb_stereo_ba, tb_conj_ba, W_len_ba, b_len_ba,
           W_angle, b_angle):
    N = idx_atomic.shape[0]
    E = idx_btype.shape[0]
    E2 = bond_angle.shape[0]
    B = 10000
    nb_b, nb_g = E // B, E2 // B

    Wb = jnp.concatenate(
        [W_mass, b_mass[None, :], jnp.zeros((3, 128), jnp.float32)], axis=0)
    tc_pad = t_chir.astype(jnp.float32)
    td_pad = jnp.concatenate(
        [t_deg, jnp.zeros((4, 128), jnp.float32)], axis=0)
    lut, tcd = pl.pallas_call(
        _prep_body,
        in_specs=[pl.BlockSpec((24, 128), lambda: (0, 0)),
                  pl.BlockSpec((8, 128), lambda: (0, 0)),
                  pl.BlockSpec((16, 128), lambda: (0, 0))],
        out_specs=[pl.BlockSpec((_LUTROWS, 128), lambda: (0, 0)),
                   pl.BlockSpec((96, 128), lambda: (0, 0))],
        out_shape=[jax.ShapeDtypeStruct((_LUTROWS, 128), jnp.float32),
                   jax.ShapeDtypeStruct((96, 128), jnp.float32)],
    )(Wb, tc_pad, td_pad)

    ta_flat = jnp.concatenate(
        [t_atomic.astype(jnp.float32), tcd, t_charge.astype(jnp.float32)],
        axis=0).reshape(-1)
    lut_flat = lut.reshape(-1)

    sc_call = pl.kernel(
        _sc_atom,
        out_type=jax.ShapeDtypeStruct((N, 128), jnp.float32),
        mesh=plsc.VectorSubcoreMesh(core_axis_name="c", subcore_axis_name="s"),
        compiler_params=pltpu.CompilerParams(needs_layout_passes=False),
        scratch_types=[
            pltpu.VMEM((_CHUNK,), jnp.int32),
            pltpu.VMEM((_CHUNK,), jnp.int32),
            pltpu.VMEM((_CHUNK,), jnp.int32),
            pltpu.VMEM((_CHUNK,), jnp.int32),
            pltpu.VMEM((_CHUNK,), jnp.float32),
            pltpu.VMEM((231 * 128,), jnp.float32),
            pltpu.VMEM((_LUTROWS * 128,), jnp.float32),
            pltpu.VMEM((_PIECE, 128), jnp.float32),
            pltpu.VMEM((_PIECE, 128), jnp.float32),
            pltpu.SemaphoreType.DMA,
            pltpu.SemaphoreType.DMA,
        ],
    )
    atom_feats = sc_call(
        idx_atomic.astype(jnp.int32), idx_chir.astype(jnp.int32),
        idx_deg.astype(jnp.int32), idx_charge.astype(jnp.int32),
        atom_mass.astype(jnp.float32), ta_flat, lut_flat)

    z7 = jnp.zeros((7, 128), jnp.float32)
    T_ab = jnp.concatenate(
        [tb_type_ab, tb_stereo_ab, tb_conj_ab, W_len_ab, b_len_ab[None, :], z7],
        axis=0).astype(jnp.bfloat16)
    T_ba = jnp.concatenate(
        [tb_type_ba, tb_stereo_ba, tb_conj_ba, W_len_ba, b_len_ba[None, :], z7],
        axis=0).astype(jnp.bfloat16)
    T_ang = jnp.concatenate(
        [W_angle, b_angle[None, :], z7], axis=0).astype(jnp.bfloat16)

    it3 = idx_btype.astype(jnp.int32).reshape(nb_b, 1, B)
    is3 = idx_bstereo.astype(jnp.int32).reshape(nb_b, 1, B)
    ij3 = idx_bconj.astype(jnp.int32).reshape(nb_b, 1, B)
    ln3 = bond_length.astype(jnp.float32).reshape(nb_b, 1, B)

    bond_attr_ab, bond_node_ba = pl.pallas_call(
        _bond_body,
        grid=(nb_b,),
        in_specs=[_vec_spec(B)] * 4 + [_tab_spec((48, 128))] * 2,
        out_specs=[pl.BlockSpec((B, 128), lambda i: (i, 0))] * 2,
        out_shape=[jax.ShapeDtypeStruct((E, 128), jnp.float32)] * 2,
    )(it3, is3, ij3, ln3, T_ab, T_ba)

    ag3 = bond_angle.astype(jnp.float32).reshape(nb_g, 1, B)
    angle_attr = pl.pallas_call(
        _angle_body,
        grid=(nb_g,),
        in_specs=[_vec_spec(B), _tab_spec((40, 128))],
        out_specs=pl.BlockSpec((B, 128), lambda i: (i, 0)),
        out_shape=jax.ShapeDtypeStruct((E2, 128), jnp.float32),
    )(ag3, T_ang)

    return (atom_feats, bond_attr_ab, ab_edge_index, bond_node_ba,
            angle_attr, ba_edge_index)


# speedup vs baseline: 1.5381x; 1.5381x over previous
import functools

import jax
import jax.numpy as jnp
from jax import lax
from jax.experimental import pallas as pl
from jax.experimental.pallas import tpu as pltpu
from jax.experimental.pallas import tpu_sc as plsc

_GAMMA = 10.0
_NA = 50000
_CHUNK = 1568
_PIECE = 112
_LUTB = 384
_LUTROWS = 392


def _sc_atom(ia_h, ic_h, id_h, iq_h, m_h, ta_h, lut_h, out_h,
             ia_v, ic_v, id_v, iq_v, m_v, ta_v, lut_v, ob0, ob1, sem0, sem1):
    w = lax.axis_index("s") * 2 + lax.axis_index("c")
    base = jnp.minimum(w * _CHUNK, _NA - _CHUNK)
    pltpu.sync_copy(ta_h, ta_v)
    pltpu.sync_copy(lut_h, lut_v)
    pltpu.sync_copy(ia_h.at[pl.ds(base, _CHUNK)], ia_v)
    pltpu.sync_copy(ic_h.at[pl.ds(base, _CHUNK)], ic_v)
    pltpu.sync_copy(id_h.at[pl.ds(base, _CHUNK)], id_v)
    pltpu.sync_copy(iq_h.at[pl.ds(base, _CHUNK)], iq_v)
    pltpu.sync_copy(m_h.at[pl.ds(base, _CHUNK)], m_v)
    lane = lax.iota(jnp.int32, 16)
    copies = []
    for p in range(_CHUNK // _PIECE):
        ob = ob0 if p % 2 == 0 else ob1
        sem = sem0 if p % 2 == 0 else sem1
        if p >= 2:
            copies[p - 2].wait()

        def group(g, carry, p=p, ob=ob):
            off = p * _PIECE + g * 16
            idx16 = lane + off
            ia16 = plsc.load_gather(ia_v, [idx16])
            ic16 = plsc.load_gather(ic_v, [idx16])
            id16 = plsc.load_gather(id_v, [idx16])
            iq16 = plsc.load_gather(iq_v, [idx16])
            m16 = plsc.load_gather(m_v, [idx16])
            abase = ia16 * 128
            cdbase = (119 + ic16 * 12 + id16) * 128
            qbase = (215 + iq16) * 128
            xs = m16 * (_LUTB / 20.0)
            qq = jnp.maximum(jnp.minimum(xs.astype(jnp.int32), _LUTB - 1), 0)
            fr = xs - qq.astype(jnp.float32)
            lbase = qq * 128
            row = g * 16 + lane

            @plsc.parallel_loop(0, 128, unroll=8)
            def colstep(j):
                acc = plsc.load_gather(ta_v, [abase + j])
                acc = acc + plsc.load_gather(ta_v, [cdbase + j])
                acc = acc + plsc.load_gather(ta_v, [qbase + j])
                la = plsc.load_gather(lut_v, [lbase + j])
                lb = plsc.load_gather(lut_v, [lbase + 128 + j])
                acc = acc + la + fr * (lb - la)
                plsc.store_scatter(ob, [row, lane * 0 + j], acc)

            return carry

        lax.fori_loop(0, _PIECE // 16, group, 0)
        copies.append(pltpu.async_copy(
            ob, out_h.at[pl.ds(base + p * _PIECE, _PIECE)], sem))
    copies[-2].wait()
    copies[-1].wait()


def _prep_body(Wb, tc, td, lut, tcd):
    colr = lax.broadcasted_iota(jnp.int32, (24, 1), 0).astype(jnp.float32)
    ctr = jnp.where(colr < 20.0, colr, -1e6)
    xs = lax.broadcasted_iota(jnp.int32, (1, _LUTROWS), 1).astype(jnp.float32)
    xs = xs * (20.0 / _LUTB)
    F = jnp.where(colr == 20.0, 1.0, jnp.exp(-_GAMMA * (xs - ctr) ** 2))
    lut[...] = lax.dot_general(F, Wb[...], (((0,), (0,)), ((), ())),
                               preferred_element_type=jnp.float32)
    r = lax.broadcasted_iota(jnp.int32, (1, 96), 1)
    rc = r // 12
    rd = r - rc * 12
    ohc = (lax.broadcasted_iota(jnp.int32, (8, 1), 0) == rc).astype(jnp.float32)
    ohd = (lax.broadcasted_iota(jnp.int32, (16, 1), 0) == rd).astype(jnp.float32)
    tcd[...] = (
        lax.dot_general(ohc, tc[...], (((0,), (0,)), ((), ())),
                        preferred_element_type=jnp.float32)
        + lax.dot_general(ohd, td[...], (((0,), (0,)), ((), ())),
                          preferred_element_type=jnp.float32))


def _bond_body(it, ist, ij, ln, Tab, Tba, oab, oba):
    one = jnp.bfloat16(1.0)
    zero = jnp.bfloat16(0.0)
    colh = lax.broadcasted_iota(jnp.int32, (16, 1), 0).astype(jnp.bfloat16)
    itb = it[0, 0, :][None, :].astype(jnp.bfloat16)
    isb = ist[0, 0, :][None, :].astype(jnp.bfloat16) + 8
    FH = jnp.where((colh == itb) | (colh == isb), one, zero)
    colr = lax.broadcasted_iota(jnp.int32, (32, 1), 0).astype(jnp.float32) + 16.0
    ctr = jnp.where((colr >= 20.0) & (colr < 40.0), (colr - 20.0) * 0.1, -1e6)
    x = ln[0, 0, :][None, :]
    t = (-_GAMMA * (x - ctr) ** 2).astype(jnp.bfloat16)
    hot = (colr == ij[0, 0, :][None, :].astype(jnp.float32) + 16.0) | (colr == 40.0)
    FR = jnp.where(hot, one, jnp.exp(t))
    F = jnp.concatenate([FH, FR], axis=0)
    oab[...] = lax.dot_general(F, Tab[...], (((0,), (0,)), ((), ())),
                               preferred_element_type=jnp.float32)
    oba[...] = lax.dot_general(F, Tba[...], (((0,), (0,)), ((), ())),
                               preferred_element_type=jnp.float32)


def _angle_body(ang, T, o):
    one = jnp.bfloat16(1.0)
    col = lax.broadcasted_iota(jnp.int32, (40, 1), 0).astype(jnp.float32)
    ctr = jnp.where(col < 32.0, col * 0.1, -1e6)
    x = ang[0, 0, :][None, :]
    t = (-_GAMMA * (x - ctr) ** 2).astype(jnp.bfloat16)
    F = jnp.where(col == 32.0, one, jnp.exp(t))
    o[...] = lax.dot_general(F, T[...], (((0,), (0,)), ((), ())),
                             preferred_element_type=jnp.float32)


def _vec_spec(B):
    return pl.BlockSpec((1, 1, B), lambda i: (i, 0, 0))


def _tab_spec(shape):
    return pl.BlockSpec(shape, lambda i: (0, 0))


def kernel(idx_atomic, idx_chir, idx_deg, idx_charge, atom_mass, idx_btype,
           idx_bstereo, idx_bconj, bond_length, ab_edge_index, bond_angle,
           ba_edge_index, t_atomic, t_chir, t_deg, t_charge, W_mass, b_mass,
           tb_type_ab, tb_stereo_ab, tb_conj_ab, W_len_ab, b_len_ab,
           tb_type_ba, tb_stereo_ba, tb_conj_ba, W_len_ba, b_len_ba,
           W_angle, b_angle):
    N = idx_atomic.shape[0]
    E = idx_btype.shape[0]
    E2 = bond_angle.shape[0]
    B = 10000
    nb_b, nb_g = E // B, E2 // B

    Wb = jnp.concatenate(
        [W_mass, b_mass[None, :], jnp.zeros((3, 128), jnp.float32)], axis=0)
    tc_pad = t_chir.astype(jnp.float32)
    td_pad = jnp.concatenate(
        [t_deg, jnp.zeros((4, 128), jnp.float32)], axis=0)
    lut, tcd = pl.pallas_call(
        _prep_body,
        in_specs=[pl.BlockSpec((24, 128), lambda: (0, 0)),
                  pl.BlockSpec((8, 128), lambda: (0, 0)),
                  pl.BlockSpec((16, 128), lambda: (0, 0))],
        out_specs=[pl.BlockSpec((_LUTROWS, 128), lambda: (0, 0)),
                   pl.BlockSpec((96, 128), lambda: (0, 0))],
        out_shape=[jax.ShapeDtypeStruct((_LUTROWS, 128), jnp.float32),
                   jax.ShapeDtypeStruct((96, 128), jnp.float32)],
    )(Wb, tc_pad, td_pad)

    ta_flat = jnp.concatenate(
        [t_atomic.astype(jnp.float32), tcd, t_charge.astype(jnp.float32)],
        axis=0).reshape(-1)
    lut_flat = lut.reshape(-1)

    sc_call = pl.kernel(
        _sc_atom,
        out_type=jax.ShapeDtypeStruct((N, 128), jnp.float32),
        mesh=plsc.VectorSubcoreMesh(core_axis_name="c", subcore_axis_name="s"),
        compiler_params=pltpu.CompilerParams(needs_layout_passes=False),
        scratch_types=[
            pltpu.VMEM((_CHUNK,), jnp.int32),
            pltpu.VMEM((_CHUNK,), jnp.int32),
            pltpu.VMEM((_CHUNK,), jnp.int32),
            pltpu.VMEM((_CHUNK,), jnp.int32),
            pltpu.VMEM((_CHUNK,), jnp.float32),
            pltpu.VMEM((231 * 128,), jnp.float32),
            pltpu.VMEM((_LUTROWS * 128,), jnp.float32),
            pltpu.VMEM((_PIECE, 128), jnp.float32),
            pltpu.VMEM((_PIECE, 128), jnp.float32),
            pltpu.SemaphoreType.DMA,
            pltpu.SemaphoreType.DMA,
        ],
    )
    atom_feats = sc_call(
        idx_atomic.astype(jnp.int32), idx_chir.astype(jnp.int32),
        idx_deg.astype(jnp.int32), idx_charge.astype(jnp.int32),
        atom_mass.astype(jnp.float32), ta_flat, lut_flat)

    z7 = jnp.zeros((7, 128), jnp.float32)
    T_ab = jnp.concatenate(
        [tb_type_ab, tb_stereo_ab, tb_conj_ab, W_len_ab, b_len_ab[None, :], z7],
        axis=0).astype(jnp.bfloat16)
    T_ba = jnp.concatenate(
        [tb_type_ba, tb_stereo_ba, tb_conj_ba, W_len_ba, b_len_ba[None, :], z7],
        axis=0).astype(jnp.bfloat16)
    T_ang = jnp.concatenate(
        [W_angle, b_angle[None, :], z7], axis=0).astype(jnp.bfloat16)

    it3 = idx_btype.astype(jnp.int32).reshape(nb_b, 1, B)
    is3 = idx_bstereo.astype(jnp.int32).reshape(nb_b, 1, B)
    ij3 = idx_bconj.astype(jnp.int32).reshape(nb_b, 1, B)
    ln3 = bond_length.astype(jnp.float32).reshape(nb_b, 1, B)

    bond_attr_ab, bond_node_ba = pl.pallas_call(
        _bond_body,
        grid=(nb_b,),
        in_specs=[_vec_spec(B)] * 4 + [_tab_spec((48, 128))] * 2,
        out_specs=[pl.BlockSpec((B, 128), lambda i: (i, 0))] * 2,
        out_shape=[jax.ShapeDtypeStruct((E, 128), jnp.float32)] * 2,
    )(it3, is3, ij3, ln3, T_ab, T_ba)

    ag3 = bond_angle.astype(jnp.float32).reshape(nb_g, 1, B)
    angle_attr = pl.pallas_call(
        _angle_body,
        grid=(nb_g,),
        in_specs=[_vec_spec(B), _tab_spec((40, 128))],
        out_specs=pl.BlockSpec((B, 128), lambda i: (i, 0)),
        out_shape=jax.ShapeDtypeStruct((E2, 128), jnp.float32),
    )(ag3, T_ang)

    return (atom_feats, bond_attr_ab, ab_edge_index, bond_node_ba,
            angle_attr, ba_edge_index)


# speedup vs baseline: 4.5887x; 2.9833x over previous
import functools

import jax
import jax.numpy as jnp
from jax import lax
from jax.experimental import pallas as pl
from jax.experimental.pallas import tpu as pltpu
from jax.experimental.pallas import tpu_sc as plsc

_GAMMA = 10.0
_NA = 50000
_CHUNK = 1568
_PIECE = 112
_LUTB = 256
_LUTROWS = 264


def _sc_atom(ia_h, ic_h, id_h, iq_h, m_h, ta_h, lut_h, out_h,
             ia_v, ic_v, id_v, iq_v, m_v, ta_v, lut_v, ob0, ob1, sem0, sem1):
    w = lax.axis_index("s") * 2 + lax.axis_index("c")
    base = jnp.minimum(w * _CHUNK, _NA - _CHUNK)
    pltpu.sync_copy(ta_h, ta_v)
    pltpu.sync_copy(lut_h, lut_v)
    pltpu.sync_copy(ia_h.at[pl.ds(base, _CHUNK)], ia_v)
    pltpu.sync_copy(ic_h.at[pl.ds(base, _CHUNK)], ic_v)
    pltpu.sync_copy(id_h.at[pl.ds(base, _CHUNK)], id_v)
    pltpu.sync_copy(iq_h.at[pl.ds(base, _CHUNK)], iq_v)
    pltpu.sync_copy(m_h.at[pl.ds(base, _CHUNK)], m_v)
    lane = lax.iota(jnp.int32, 16)
    copies = []
    for p in range(_CHUNK // _PIECE):
        ob = ob0 if p % 2 == 0 else ob1
        sem = sem0 if p % 2 == 0 else sem1
        if p >= 2:
            copies[p - 2].wait()

        def group(g, carry, p=p, ob=ob):
            off = p * _PIECE + g * 16
            idx16 = lane + off
            ia16 = plsc.load_gather(ia_v, [idx16])
            ic16 = plsc.load_gather(ic_v, [idx16])
            id16 = plsc.load_gather(id_v, [idx16])
            iq16 = plsc.load_gather(iq_v, [idx16])
            m16 = plsc.load_gather(m_v, [idx16])
            abase = ia16 * 129
            cdbase = (119 + ic16 * 12 + id16) * 129
            qbase = (215 + iq16) * 129
            xs = m16 * (_LUTB / 20.0)
            qq = jnp.maximum(jnp.minimum(xs.astype(jnp.int32), _LUTB - 1), 0)
            fr = xs - qq.astype(jnp.float32)
            lbase = qq * 129
            row = g * 16 + lane

            @plsc.parallel_loop(0, 128, unroll=8)
            def colstep(j):
                acc = plsc.load_gather(ta_v, [abase + j])
                acc = acc + plsc.load_gather(ta_v, [cdbase + j])
                acc = acc + plsc.load_gather(ta_v, [qbase + j])
                la = plsc.load_gather(lut_v, [lbase + j])
                lb = plsc.load_gather(lut_v, [lbase + 129 + j])
                acc = acc + la + fr * (lb - la)
                plsc.store_scatter(ob, [row, lane * 0 + j], acc)

            return carry

        lax.fori_loop(0, _PIECE // 16, group, 0)
        copies.append(pltpu.async_copy(
            ob.at[:, pl.ds(0, 128)],
            out_h.at[pl.ds(base + p * _PIECE, _PIECE)], sem))
    copies[-2].wait()
    copies[-1].wait()


def _prep_body(Wb, tc, td, lut, tcd):
    colr = lax.broadcasted_iota(jnp.int32, (24, 1), 0).astype(jnp.float32)
    ctr = jnp.where(colr < 20.0, colr, -1e6)
    xs = lax.broadcasted_iota(jnp.int32, (1, _LUTROWS), 1).astype(jnp.float32)
    xs = xs * (20.0 / _LUTB)
    F = jnp.where(colr == 20.0, 1.0, jnp.exp(-_GAMMA * (xs - ctr) ** 2))
    lut[...] = lax.dot_general(F, Wb[...], (((0,), (0,)), ((), ())),
                               preferred_element_type=jnp.float32)
    r = lax.broadcasted_iota(jnp.int32, (1, 96), 1)
    rc = r // 12
    rd = r - rc * 12
    ohc = (lax.broadcasted_iota(jnp.int32, (8, 1), 0) == rc).astype(jnp.float32)
    ohd = (lax.broadcasted_iota(jnp.int32, (16, 1), 0) == rd).astype(jnp.float32)
    tcd[...] = (
        lax.dot_general(ohc, tc[...], (((0,), (0,)), ((), ())),
                        preferred_element_type=jnp.float32)
        + lax.dot_general(ohd, td[...], (((0,), (0,)), ((), ())),
                          preferred_element_type=jnp.float32))


def _bond_body(it, ist, ij, ln, Tab, Tba, oab, oba):
    one = jnp.bfloat16(1.0)
    zero = jnp.bfloat16(0.0)
    colh = lax.broadcasted_iota(jnp.int32, (16, 1), 0).astype(jnp.bfloat16)
    itb = it[0, 0, :][None, :].astype(jnp.bfloat16)
    isb = ist[0, 0, :][None, :].astype(jnp.bfloat16) + 8
    FH = jnp.where((colh == itb) | (colh == isb), one, zero)
    colr = lax.broadcasted_iota(jnp.int32, (32, 1), 0).astype(jnp.float32) + 16.0
    ctr = jnp.where((colr >= 20.0) & (colr < 40.0), (colr - 20.0) * 0.1, -1e6)
    x = ln[0, 0, :][None, :]
    t = (-_GAMMA * (x - ctr) ** 2).astype(jnp.bfloat16)
    hot = (colr == ij[0, 0, :][None, :].astype(jnp.float32) + 16.0) | (colr == 40.0)
    FR = jnp.where(hot, one, jnp.exp(t))
    F = jnp.concatenate([FH, FR], axis=0)
    oab[...] = lax.dot_general(F, Tab[...], (((0,), (0,)), ((), ())),
                               preferred_element_type=jnp.float32)
    oba[...] = lax.dot_general(F, Tba[...], (((0,), (0,)), ((), ())),
                               preferred_element_type=jnp.float32)


def _angle_body(ang, T, o):
    one = jnp.bfloat16(1.0)
    col = lax.broadcasted_iota(jnp.int32, (40, 1), 0).astype(jnp.float32)
    ctr = jnp.where(col < 32.0, col * 0.1, -1e6)
    x = ang[0, 0, :][None, :]
    t = (-_GAMMA * (x - ctr) ** 2).astype(jnp.bfloat16)
    F = jnp.where(col == 32.0, one, jnp.exp(t))
    o[...] = lax.dot_general(F, T[...], (((0,), (0,)), ((), ())),
                             preferred_element_type=jnp.float32)


def _vec_spec(B):
    return pl.BlockSpec((1, 1, B), lambda i: (i, 0, 0))


def _tab_spec(shape):
    return pl.BlockSpec(shape, lambda i: (0, 0))


def kernel(idx_atomic, idx_chir, idx_deg, idx_charge, atom_mass, idx_btype,
           idx_bstereo, idx_bconj, bond_length, ab_edge_index, bond_angle,
           ba_edge_index, t_atomic, t_chir, t_deg, t_charge, W_mass, b_mass,
           tb_type_ab, tb_stereo_ab, tb_conj_ab, W_len_ab, b_len_ab,
           tb_type_ba, tb_stereo_ba, tb_conj_ba, W_len_ba, b_len_ba,
           W_angle, b_angle):
    N = idx_atomic.shape[0]
    E = idx_btype.shape[0]
    E2 = bond_angle.shape[0]
    B = 10000
    nb_b, nb_g = E // B, E2 // B

    Wb = jnp.concatenate(
        [W_mass, b_mass[None, :], jnp.zeros((3, 128), jnp.float32)], axis=0)
    tc_pad = t_chir.astype(jnp.float32)
    td_pad = jnp.concatenate(
        [t_deg, jnp.zeros((4, 128), jnp.float32)], axis=0)
    lut, tcd = pl.pallas_call(
        _prep_body,
        in_specs=[pl.BlockSpec((24, 128), lambda: (0, 0)),
                  pl.BlockSpec((8, 128), lambda: (0, 0)),
                  pl.BlockSpec((16, 128), lambda: (0, 0))],
        out_specs=[pl.BlockSpec((_LUTROWS, 128), lambda: (0, 0)),
                   pl.BlockSpec((96, 128), lambda: (0, 0))],
        out_shape=[jax.ShapeDtypeStruct((_LUTROWS, 128), jnp.float32),
                   jax.ShapeDtypeStruct((96, 128), jnp.float32)],
    )(Wb, tc_pad, td_pad)

    ta_flat = jnp.pad(jnp.concatenate(
        [t_atomic.astype(jnp.float32), tcd, t_charge.astype(jnp.float32)],
        axis=0), ((0, 0), (0, 1))).reshape(-1)
    lut_flat = jnp.pad(lut, ((0, 0), (0, 1))).reshape(-1)

    sc_call = pl.kernel(
        _sc_atom,
        out_type=jax.ShapeDtypeStruct((N, 128), jnp.float32),
        mesh=plsc.VectorSubcoreMesh(core_axis_name="c", subcore_axis_name="s"),
        compiler_params=pltpu.CompilerParams(needs_layout_passes=False),
        scratch_types=[
            pltpu.VMEM((_CHUNK,), jnp.int32),
            pltpu.VMEM((_CHUNK,), jnp.int32),
            pltpu.VMEM((_CHUNK,), jnp.int32),
            pltpu.VMEM((_CHUNK,), jnp.int32),
            pltpu.VMEM((_CHUNK,), jnp.float32),
            pltpu.VMEM((231 * 129,), jnp.float32),
            pltpu.VMEM((_LUTROWS * 129,), jnp.float32),
            pltpu.VMEM((_PIECE, 129), jnp.float32),
            pltpu.VMEM((_PIECE, 129), jnp.float32),
            pltpu.SemaphoreType.DMA,
            pltpu.SemaphoreType.DMA,
        ],
    )
    atom_feats = sc_call(
        idx_atomic.astype(jnp.int32), idx_chir.astype(jnp.int32),
        idx_deg.astype(jnp.int32), idx_charge.astype(jnp.int32),
        atom_mass.astype(jnp.float32), ta_flat, lut_flat)

    z7 = jnp.zeros((7, 128), jnp.float32)
    T_ab = jnp.concatenate(
        [tb_type_ab, tb_stereo_ab, tb_conj_ab, W_len_ab, b_len_ab[None, :], z7],
        axis=0).astype(jnp.bfloat16)
    T_ba = jnp.concatenate(
        [tb_type_ba, tb_stereo_ba, tb_conj_ba, W_len_ba, b_len_ba[None, :], z7],
        axis=0).astype(jnp.bfloat16)
    T_ang = jnp.concatenate(
        [W_angle, b_angle[None, :], z7], axis=0).astype(jnp.bfloat16)

    it3 = idx_btype.astype(jnp.int32).reshape(nb_b, 1, B)
    is3 = idx_bstereo.astype(jnp.int32).reshape(nb_b, 1, B)
    ij3 = idx_bconj.astype(jnp.int32).reshape(nb_b, 1, B)
    ln3 = bond_length.astype(jnp.float32).reshape(nb_b, 1, B)

    bond_attr_ab, bond_node_ba = pl.pallas_call(
        _bond_body,
        grid=(nb_b,),
        in_specs=[_vec_spec(B)] * 4 + [_tab_spec((48, 128))] * 2,
        out_specs=[pl.BlockSpec((B, 128), lambda i: (i, 0))] * 2,
        out_shape=[jax.ShapeDtypeStruct((E, 128), jnp.float32)] * 2,
    )(it3, is3, ij3, ln3, T_ab, T_ba)

    ag3 = bond_angle.astype(jnp.float32).reshape(nb_g, 1, B)
    angle_attr = pl.pallas_call(
        _angle_body,
        grid=(nb_g,),
        in_specs=[_vec_spec(B), _tab_spec((40, 128))],
        out_specs=pl.BlockSpec((B, 128), lambda i: (i, 0)),
        out_shape=jax.ShapeDtypeStruct((E2, 128), jnp.float32),
    )(ag3, T_ang)

    return (atom_feats, bond_attr_ab, ab_edge_index, bond_node_ba,
            angle_attr, ba_edge_index)


# speedup vs baseline: 5.3832x; 1.1732x over previous
import functools

import jax
import jax.numpy as jnp
from jax import lax
from jax.experimental import pallas as pl
from jax.experimental.pallas import tpu as pltpu
from jax.experimental.pallas import tpu_sc as plsc

_GAMMA = 10.0
_NS = 25000
_CHUNK = 784
_PIECE = 112
_LUTB = 256
_LUTROWS = 264


def _sc_atom(ia_h, ic_h, id_h, iq_h, m_h, ta_h, lut_h, out_h,
             ia_v, ic_v, id_v, iq_v, m_v, ta_v, lut_v, ob0, ob1, sem0, sem1):
    w = lax.axis_index("s") * 2 + lax.axis_index("c")
    base = jnp.minimum(w * _CHUNK, _NS - _CHUNK)
    pltpu.sync_copy(ta_h, ta_v)
    pltpu.sync_copy(lut_h, lut_v)
    pltpu.sync_copy(ia_h.at[pl.ds(base, _CHUNK)], ia_v)
    pltpu.sync_copy(ic_h.at[pl.ds(base, _CHUNK)], ic_v)
    pltpu.sync_copy(id_h.at[pl.ds(base, _CHUNK)], id_v)
    pltpu.sync_copy(iq_h.at[pl.ds(base, _CHUNK)], iq_v)
    pltpu.sync_copy(m_h.at[pl.ds(base, _CHUNK)], m_v)
    lane = lax.iota(jnp.int32, 16)
    copies = []
    for p in range(_CHUNK // _PIECE):
        ob = ob0 if p % 2 == 0 else ob1
        sem = sem0 if p % 2 == 0 else sem1
        if p >= 2:
            copies[p - 2].wait()

        def group(g, carry, p=p, ob=ob):
            off = p * _PIECE + g * 16
            idx16 = lane + off
            ia16 = plsc.load_gather(ia_v, [idx16])
            ic16 = plsc.load_gather(ic_v, [idx16])
            id16 = plsc.load_gather(id_v, [idx16])
            iq16 = plsc.load_gather(iq_v, [idx16])
            m16 = plsc.load_gather(m_v, [idx16])
            abase = ia16 * 129
            cdbase = (119 + ic16 * 12 + id16) * 129
            qbase = (215 + iq16) * 129
            xs = m16 * (_LUTB / 20.0)
            qq = jnp.maximum(jnp.minimum(xs.astype(jnp.int32), _LUTB - 1), 0)
            fr = xs - qq.astype(jnp.float32)
            lbase = qq * 129
            row = g * 16 + lane

            @plsc.parallel_loop(0, 128, unroll=8)
            def colstep(j):
                acc = plsc.load_gather(ta_v, [abase + j])
                acc = acc + plsc.load_gather(ta_v, [cdbase + j])
                acc = acc + plsc.load_gather(ta_v, [qbase + j])
                la = plsc.load_gather(lut_v, [lbase + j])
                lb = plsc.load_gather(lut_v, [lbase + 129 + j])
                acc = acc + la + fr * (lb - la)
                plsc.store_scatter(ob, [row, lane * 0 + j], acc)

            return carry

        lax.fori_loop(0, _PIECE // 16, group, 0)
        copies.append(pltpu.async_copy(
            ob.at[:, pl.ds(0, 128)],
            out_h.at[pl.ds(base + p * _PIECE, _PIECE)], sem))
    copies[-2].wait()
    copies[-1].wait()


def _prep_body(Wb, tc, td, lut, tcd):
    colr = lax.broadcasted_iota(jnp.int32, (24, 1), 0).astype(jnp.float32)
    ctr = jnp.where(colr < 20.0, colr, -1e6)
    xs = lax.broadcasted_iota(jnp.int32, (1, _LUTROWS), 1).astype(jnp.float32)
    xs = xs * (20.0 / _LUTB)
    F = jnp.where(colr == 20.0, 1.0, jnp.exp(-_GAMMA * (xs - ctr) ** 2))
    lut[...] = lax.dot_general(F, Wb[...], (((0,), (0,)), ((), ())),
                               preferred_element_type=jnp.float32)
    r = lax.broadcasted_iota(jnp.int32, (1, 96), 1)
    rc = r // 12
    rd = r - rc * 12
    ohc = (lax.broadcasted_iota(jnp.int32, (8, 1), 0) == rc).astype(jnp.float32)
    ohd = (lax.broadcasted_iota(jnp.int32, (16, 1), 0) == rd).astype(jnp.float32)
    tcd[...] = (
        lax.dot_general(ohc, tc[...], (((0,), (0,)), ((), ())),
                        preferred_element_type=jnp.float32)
        + lax.dot_general(ohd, td[...], (((0,), (0,)), ((), ())),
                          preferred_element_type=jnp.float32))


def _atom_body(ia, ic, idg, iq, mass, T, src, o):
    one = jnp.bfloat16(1.0)
    zero = jnp.bfloat16(0.0)
    colh = lax.broadcasted_iota(jnp.int32, (160, 1), 0).astype(jnp.bfloat16)
    iab = ia[0, 0, :][None, :].astype(jnp.bfloat16)
    icb = ic[0, 0, :][None, :].astype(jnp.bfloat16) + 119
    idb = idg[0, 0, :][None, :].astype(jnp.bfloat16) + 127
    iqb = iq[0, 0, :][None, :].astype(jnp.bfloat16) + 139
    hot = (colh == iab) | (colh == icb) | (colh == idb) | (colh == iqb)
    FH = jnp.where(hot, one, zero)
    colr = lax.broadcasted_iota(jnp.int32, (32, 1), 0).astype(jnp.float32)
    ctr = jnp.where(colr < 20.0, colr, -1e6)
    x = mass[0, 0, :][None, :]
    t = (-_GAMMA * (x - ctr) ** 2).astype(jnp.bfloat16)
    FR = jnp.where(colr == 20.0, one, jnp.exp(t))
    F = jnp.concatenate([FH, FR], axis=0)
    o[...] = lax.dot_general(F, T[...], (((0,), (0,)), ((), ())),
                             preferred_element_type=jnp.float32)


def _bond_body(it, ist, ij, ln, Tab, Tba, oab, oba):
    one = jnp.bfloat16(1.0)
    zero = jnp.bfloat16(0.0)
    colh = lax.broadcasted_iota(jnp.int32, (16, 1), 0).astype(jnp.bfloat16)
    itb = it[0, 0, :][None, :].astype(jnp.bfloat16)
    isb = ist[0, 0, :][None, :].astype(jnp.bfloat16) + 8
    FH = jnp.where((colh == itb) | (colh == isb), one, zero)
    colr = lax.broadcasted_iota(jnp.int32, (32, 1), 0).astype(jnp.float32) + 16.0
    ctr = jnp.where((colr >= 20.0) & (colr < 40.0), (colr - 20.0) * 0.1, -1e6)
    x = ln[0, 0, :][None, :]
    t = (-_GAMMA * (x - ctr) ** 2).astype(jnp.bfloat16)
    hot = (colr == ij[0, 0, :][None, :].astype(jnp.float32) + 16.0) | (colr == 40.0)
    FR = jnp.where(hot, one, jnp.exp(t))
    F = jnp.concatenate([FH, FR], axis=0)
    oab[...] = lax.dot_general(F, Tab[...], (((0,), (0,)), ((), ())),
                               preferred_element_type=jnp.float32)
    oba[...] = lax.dot_general(F, Tba[...], (((0,), (0,)), ((), ())),
                               preferred_element_type=jnp.float32)


def _angle_body(ang, T, o):
    one = jnp.bfloat16(1.0)
    col = lax.broadcasted_iota(jnp.int32, (40, 1), 0).astype(jnp.float32)
    ctr = jnp.where(col < 32.0, col * 0.1, -1e6)
    x = ang[0, 0, :][None, :]
    t = (-_GAMMA * (x - ctr) ** 2).astype(jnp.bfloat16)
    F = jnp.where(col == 32.0, one, jnp.exp(t))
    o[...] = lax.dot_general(F, T[...], (((0,), (0,)), ((), ())),
                             preferred_element_type=jnp.float32)


def _vec_spec(B):
    return pl.BlockSpec((1, 1, B), lambda i: (i, 0, 0))


def _tab_spec(shape):
    return pl.BlockSpec(shape, lambda i: (0, 0))


def kernel(idx_atomic, idx_chir, idx_deg, idx_charge, atom_mass, idx_btype,
           idx_bstereo, idx_bconj, bond_length, ab_edge_index, bond_angle,
           ba_edge_index, t_atomic, t_chir, t_deg, t_charge, W_mass, b_mass,
           tb_type_ab, tb_stereo_ab, tb_conj_ab, W_len_ab, b_len_ab,
           tb_type_ba, tb_stereo_ba, tb_conj_ba, W_len_ba, b_len_ba,
           W_angle, b_angle):
    N = idx_atomic.shape[0]
    E = idx_btype.shape[0]
    E2 = bond_angle.shape[0]
    B = 10000
    nb_b, nb_g = E // B, E2 // B

    Wb = jnp.concatenate(
        [W_mass, b_mass[None, :], jnp.zeros((3, 128), jnp.float32)], axis=0)
    tc_pad = t_chir.astype(jnp.float32)
    td_pad = jnp.concatenate(
        [t_deg, jnp.zeros((4, 128), jnp.float32)], axis=0)
    lut, tcd = pl.pallas_call(
        _prep_body,
        in_specs=[pl.BlockSpec((24, 128), lambda: (0, 0)),
                  pl.BlockSpec((8, 128), lambda: (0, 0)),
                  pl.BlockSpec((16, 128), lambda: (0, 0))],
        out_specs=[pl.BlockSpec((_LUTROWS, 128), lambda: (0, 0)),
                   pl.BlockSpec((96, 128), lambda: (0, 0))],
        out_shape=[jax.ShapeDtypeStruct((_LUTROWS, 128), jnp.float32),
                   jax.ShapeDtypeStruct((96, 128), jnp.float32)],
    )(Wb, tc_pad, td_pad)

    ta_flat = jnp.pad(jnp.concatenate(
        [t_atomic.astype(jnp.float32), tcd, t_charge.astype(jnp.float32)],
        axis=0), ((0, 0), (0, 1))).reshape(-1)
    lut_flat = jnp.pad(lut, ((0, 0), (0, 1))).reshape(-1)

    sc_call = pl.kernel(
        _sc_atom,
        out_type=jax.ShapeDtypeStruct((N, 128), jnp.float32),
        mesh=plsc.VectorSubcoreMesh(core_axis_name="c", subcore_axis_name="s"),
        compiler_params=pltpu.CompilerParams(needs_layout_passes=False),
        scratch_types=[
            pltpu.VMEM((_CHUNK,), jnp.int32),
            pltpu.VMEM((_CHUNK,), jnp.int32),
            pltpu.VMEM((_CHUNK,), jnp.int32),
            pltpu.VMEM((_CHUNK,), jnp.int32),
            pltpu.VMEM((_CHUNK,), jnp.float32),
            pltpu.VMEM((231 * 129,), jnp.float32),
            pltpu.VMEM((_LUTROWS * 129,), jnp.float32),
            pltpu.VMEM((_PIECE, 129), jnp.float32),
            pltpu.VMEM((_PIECE, 129), jnp.float32),
            pltpu.SemaphoreType.DMA,
            pltpu.SemaphoreType.DMA,
        ],
    )
    atom_sc = sc_call(
        idx_atomic.astype(jnp.int32), idx_chir.astype(jnp.int32),
        idx_deg.astype(jnp.int32), idx_charge.astype(jnp.int32),
        atom_mass.astype(jnp.float32), ta_flat, lut_flat)

    Ba = 5000
    nb_a, sc_blocks = N // Ba, _NS // Ba
    z5 = jnp.zeros((5, 128), jnp.float32)
    z11 = jnp.zeros((11, 128), jnp.float32)
    T_atom = jnp.concatenate(
        [t_atomic, t_chir, t_deg, t_charge, z5, W_mass, b_mass[None, :], z11],
        axis=0).astype(jnp.bfloat16)
    ia3 = idx_atomic.astype(jnp.int32).reshape(nb_a, 1, Ba)
    ic3 = idx_chir.astype(jnp.int32).reshape(nb_a, 1, Ba)
    id3 = idx_deg.astype(jnp.int32).reshape(nb_a, 1, Ba)
    iq3 = idx_charge.astype(jnp.int32).reshape(nb_a, 1, Ba)
    m3 = atom_mass.astype(jnp.float32).reshape(nb_a, 1, Ba)
    atom_feats = pl.pallas_call(
        _atom_body,
        grid=(nb_a - sc_blocks,),
        in_specs=[pl.BlockSpec((1, 1, Ba), lambda i: (i + sc_blocks, 0, 0))] * 5
        + [_tab_spec((192, 128)),
           pl.BlockSpec(memory_space=pl.ANY)],
        out_specs=pl.BlockSpec((Ba, 128), lambda i: (i + sc_blocks, 0)),
        out_shape=jax.ShapeDtypeStruct((N, 128), jnp.float32),
        input_output_aliases={6: 0},
    )(ia3, ic3, id3, iq3, m3, T_atom, atom_sc)

    z7 = jnp.zeros((7, 128), jnp.float32)
    T_ab = jnp.concatenate(
        [tb_type_ab, tb_stereo_ab, tb_conj_ab, W_len_ab, b_len_ab[None, :], z7],
        axis=0).astype(jnp.bfloat16)
    T_ba = jnp.concatenate(
        [tb_type_ba, tb_stereo_ba, tb_conj_ba, W_len_ba, b_len_ba[None, :], z7],
        axis=0).astype(jnp.bfloat16)
    T_ang = jnp.concatenate(
        [W_angle, b_angle[None, :], z7], axis=0).astype(jnp.bfloat16)

    it3 = idx_btype.astype(jnp.int32).reshape(nb_b, 1, B)
    is3 = idx_bstereo.astype(jnp.int32).reshape(nb_b, 1, B)
    ij3 = idx_bconj.astype(jnp.int32).reshape(nb_b, 1, B)
    ln3 = bond_length.astype(jnp.float32).reshape(nb_b, 1, B)

    bond_attr_ab, bond_node_ba = pl.pallas_call(
        _bond_body,
        grid=(nb_b,),
        in_specs=[_vec_spec(B)] * 4 + [_tab_spec((48, 128))] * 2,
        out_specs=[pl.BlockSpec((B, 128), lambda i: (i, 0))] * 2,
        out_shape=[jax.ShapeDtypeStruct((E, 128), jnp.float32)] * 2,
    )(it3, is3, ij3, ln3, T_ab, T_ba)

    ag3 = bond_angle.astype(jnp.float32).reshape(nb_g, 1, B)
    angle_attr = pl.pallas_call(
        _angle_body,
        grid=(nb_g,),
        in_specs=[_vec_spec(B), _tab_spec((40, 128))],
        out_specs=pl.BlockSpec((B, 128), lambda i: (i, 0)),
        out_shape=jax.ShapeDtypeStruct((E2, 128), jnp.float32),
    )(ag3, T_ang)

    return (atom_feats, bond_attr_ab, ab_edge_index, bond_node_ba,
            angle_attr, ba_edge_index)
